# Initial kernel scaffold; baseline (speedup 1.0000x reference)
import jax
import jax.numpy as jnp
from jax.experimental import pallas as pl


def kernel(x, edge_index, edge_attr, state, batch, bond_batch, params):
    raise NotImplementedError("write your pallas kernel here")


# speedup vs baseline: 1.3486x; 1.3486x over previous
import jax
import jax.numpy as jnp
import numpy as np
from jax.experimental import pallas as pl


def _apply_mlp(h, layers):
    for p in layers:
        h = jax.nn.softplus(h @ p["W"] + p["b"])
    return h


def _segment_mean(vals, ids, num_segments):
    s = jax.ops.segment_sum(vals, ids, num_segments=num_segments)
    c = jax.ops.segment_sum(jnp.ones((vals.shape[0], 1), vals.dtype), ids, num_segments=num_segments)
    return s / jnp.maximum(c, 1.0)


def _megnet_module(x, edge_index, edge_attr, state, batch, bond_batch, p, skip):
    n = x.shape[0]
    b = state.shape[0]
    xe = _apply_mlp(edge_attr, p["pre_e"])
    xv = _apply_mlp(x, p["pre_v"])
    xu = _apply_mlp(state, p["pre_u"])
    src = edge_index[0]
    dst = edge_index[1]
    e_new = _apply_mlp(jnp.concatenate([xv[src], xv[dst], xe, xu[bond_batch]], axis=1), p["phi_e"])
    e_to_v = _segment_mean(e_new, dst, n)
    v_new = _apply_mlp(jnp.concatenate([xv, e_to_v, xu[batch]], axis=1), p["phi_v"])
    ue = _segment_mean(e_new, bond_batch, b)
    uv = _segment_mean(v_new, batch, b)
    u_new = _apply_mlp(jnp.concatenate([ue, uv, xu], axis=1), p["phi_u"])
    if skip:
        v_new = v_new + x
        e_new = e_new + edge_attr
        u_new = u_new + state
    return v_new, e_new, u_new


def _head_kernel(tmp_ref, w0_ref, b0_ref, w1_ref, b1_ref, w2_ref, b2_ref, o_ref):
    h = jax.nn.softplus(tmp_ref[...] @ w0_ref[...] + b0_ref[...])
    h = jax.nn.softplus(h @ w1_ref[...] + b1_ref[...])
    o_ref[...] = h @ w2_ref[...] + b2_ref[...]


def kernel(x, edge_index, edge_attr, state, batch, bond_batch, params):
    b = state.shape[0]
    x1, e1, u1 = _megnet_module(x, edge_index, edge_attr, state, batch, bond_batch, params["m1"], False)
    x2, e2, u2 = _megnet_module(x1, edge_index, e1, u1, batch, bond_batch, params["m2"], True)
    x3, e3, u3 = _megnet_module(x2, edge_index, e2, u2, batch, bond_batch, params["m3"], True)
    mv = _segment_mean(x3, batch, b)
    me = _segment_mean(e3, bond_batch, b)
    zeros = jnp.zeros_like(mv)
    tmp = jnp.concatenate([zeros, mv, zeros, me, u3], axis=1)
    hl = params["hiddens"]
    out = pl.pallas_call(
        _head_kernel,
        out_shape=jax.ShapeDtypeStruct((b, 1), jnp.float32),
    )(tmp, hl[0]["W"], hl[0]["b"], hl[1]["W"], hl[1]["b"], hl[2]["W"], hl[2]["b"])
    return out


# speedup vs baseline: 4.8543x; 3.5995x over previous
import functools

import jax
import jax.numpy as jnp
from jax import lax
from jax.experimental import pallas as pl
from jax.experimental.pallas import tpu as pltpu
from jax.experimental.pallas import tpu_sc as plsc

N_V = 50000
N_E = 800000
N_G = 512
D = 32

_E_W = 100
_E_K = 8
_E_ROWS = _E_W * _E_K
_E_CHUNKS = N_E // _E_ROWS
_N_W = 25
_N_K = 8
_N_ROWS = _N_W * _N_K
_N_CHUNKS = N_V // _N_ROWS

_NW = 32
_E_TRIPS = (_E_CHUNKS + _NW - 1) // _NW
_N_TRIPS = (_N_CHUNKS + _NW - 1) // _NW

_MESH = plsc.VectorSubcoreMesh(core_axis_name="c", subcore_axis_name="s")
_NSUB = 16
_VSUB = 10
_VROWS = N_V // _VSUB
_GROWS = N_G // _NSUB


def _worker_id():
    return lax.axis_index("s") * 2 + lax.axis_index("c")


@functools.partial(
    pl.kernel,
    out_type=(
        jax.ShapeDtypeStruct((N_E, D), jnp.float32),
        jax.ShapeDtypeStruct((N_E, D), jnp.float32),
        jax.ShapeDtypeStruct((N_E, D), jnp.float32),
        jax.ShapeDtypeStruct((N_V, D), jnp.float32),
    ),
    mesh=_MESH,
    compiler_params=pltpu.CompilerParams(use_tc_tiling_on_sc=False),
    scratch_types=[
        pltpu.VMEM_SHARED((N_V, D), jnp.float32),
        pltpu.VMEM_SHARED((N_G, D), jnp.float32),
        pltpu.VMEM((_E_K, _E_W), jnp.int32),
        pltpu.VMEM((_E_K, _E_W), jnp.int32),
        pltpu.VMEM((_E_K, _E_W), jnp.int32),
        pltpu.VMEM((_E_W, D), jnp.float32),
        pltpu.VMEM((_E_W, D), jnp.float32),
        pltpu.VMEM((_E_W, D), jnp.float32),
        pltpu.VMEM((_N_K, _N_W), jnp.int32),
        pltpu.VMEM((_N_W, D), jnp.float32),
    ],
)
def _sc_gather(xv_hbm, xu_hbm, src_hbm, dst_hbm, bb_hbm, nb_hbm,
               os_hbm, od_hbm, ob_hbm, on_hbm,
               xv_spm, xu_spm, si, di, bi, gs, gd, gb, ni, gn):
    sid = lax.axis_index("s")
    wid = _worker_id()

    @pl.when(sid < _VSUB)
    def _():
        pltpu.sync_copy(xv_hbm.at[pl.ds(sid * _VROWS, _VROWS)],
                        xv_spm.at[pl.ds(sid * _VROWS, _VROWS)])

    pltpu.sync_copy(xu_hbm.at[pl.ds(sid * _GROWS, _GROWS)],
                    xu_spm.at[pl.ds(sid * _GROWS, _GROWS)])
    plsc.subcore_barrier()

    @pl.loop(0, _E_TRIPS)
    def _(t):
        c = wid + t * _NW

        @pl.when(c < _E_CHUNKS)
        def _():
            base = c * _E_ROWS
            pltpu.sync_copy(src_hbm.at[pl.ds(c * _E_K, _E_K)], si)
            pltpu.sync_copy(dst_hbm.at[pl.ds(c * _E_K, _E_K)], di)
            pltpu.sync_copy(bb_hbm.at[pl.ds(c * _E_K, _E_K)], bi)
            for j in range(_E_K):
                w = pl.ds(base + j * _E_W, _E_W)
                pltpu.sync_copy(xv_spm.at[si.at[j]], gs)
                pltpu.sync_copy(xv_spm.at[di.at[j]], gd)
                pltpu.sync_copy(xu_spm.at[bi.at[j]], gb)
                pltpu.sync_copy(gs, os_hbm.at[w])
                pltpu.sync_copy(gd, od_hbm.at[w])
                pltpu.sync_copy(gb, ob_hbm.at[w])

    @pl.loop(0, _N_TRIPS)
    def _(t):
        c = wid + t * _NW

        @pl.when(c < _N_CHUNKS)
        def _():
            base = c * _N_ROWS
            pltpu.sync_copy(nb_hbm.at[pl.ds(c * _N_K, _N_K)], ni)
            for j in range(_N_K):
                pltpu.sync_copy(xu_spm.at[ni.at[j]], gn)
                pltpu.sync_copy(gn, on_hbm.at[pl.ds(base + j * _N_W, _N_W)])


@functools.partial(
    pl.kernel,
    out_type=(
        jax.ShapeDtypeStruct((2, N_V, D), jnp.float32),
        jax.ShapeDtypeStruct((2, N_G, D), jnp.float32),
    ),
    mesh=_MESH,
    compiler_params=pltpu.CompilerParams(use_tc_tiling_on_sc=False),
    scratch_types=[
        pltpu.VMEM_SHARED((N_V, D), jnp.float32),
        pltpu.VMEM_SHARED((N_G, D), jnp.float32),
        pltpu.VMEM((_E_K, _E_W), jnp.int32),
        pltpu.VMEM((_E_K, _E_W), jnp.int32),
        pltpu.VMEM((_E_W, D), jnp.float32),
    ],
)
def _sc_scatter_edges(e_hbm, dst_hbm, bb_hbm, z_hbm, ov_hbm, ou_hbm,
                      accv, accu, di, bi, ge):
    cid = lax.axis_index("c")
    sid = lax.axis_index("s")
    wid = _worker_id()

    @pl.when(sid < _VSUB)
    def _():
        pltpu.sync_copy(z_hbm.at[pl.ds(sid * _VROWS, _VROWS)],
                        accv.at[pl.ds(sid * _VROWS, _VROWS)])

    pltpu.sync_copy(z_hbm.at[pl.ds(sid * _GROWS, _GROWS)],
                    accu.at[pl.ds(sid * _GROWS, _GROWS)])
    plsc.subcore_barrier()

    @pl.loop(0, _E_TRIPS)
    def _(t):
        c = wid + t * _NW

        @pl.when(c < _E_CHUNKS)
        def _():
            base = c * _E_ROWS
            pltpu.sync_copy(dst_hbm.at[pl.ds(c * _E_K, _E_K)], di)
            pltpu.sync_copy(bb_hbm.at[pl.ds(c * _E_K, _E_K)], bi)
            for j in range(_E_K):
                pltpu.sync_copy(e_hbm.at[pl.ds(base + j * _E_W, _E_W)], ge)
                pltpu.sync_copy(ge, accv.at[di.at[j]], add=True)
                pltpu.sync_copy(ge, accu.at[bi.at[j]], add=True)

    plsc.subcore_barrier()

    @pl.when(sid < _VSUB)
    def _():
        pltpu.sync_copy(accv.at[pl.ds(sid * _VROWS, _VROWS)],
                        ov_hbm.at[cid, pl.ds(sid * _VROWS, _VROWS)])

    pltpu.sync_copy(accu.at[pl.ds(sid * _GROWS, _GROWS)],
                    ou_hbm.at[cid, pl.ds(sid * _GROWS, _GROWS)])


@functools.partial(
    pl.kernel,
    out_type=jax.ShapeDtypeStruct((2, N_G, D), jnp.float32),
    mesh=_MESH,
    compiler_params=pltpu.CompilerParams(use_tc_tiling_on_sc=False),
    scratch_types=[
        pltpu.VMEM_SHARED((N_G, D), jnp.float32),
        pltpu.VMEM((_N_K, _N_W), jnp.int32),
        pltpu.VMEM((_N_W, D), jnp.float32),
    ],
)
def _sc_scatter_nodes(v_hbm, nb_hbm, z_hbm, ou_hbm, accu, ni, gv):
    cid = lax.axis_index("c")
    sid = lax.axis_index("s")
    wid = _worker_id()
    pltpu.sync_copy(z_hbm.at[pl.ds(sid * _GROWS, _GROWS)],
                    accu.at[pl.ds(sid * _GROWS, _GROWS)])
    plsc.subcore_barrier()

    @pl.loop(0, _N_TRIPS)
    def _(t):
        c = wid + t * _NW

        @pl.when(c < _N_CHUNKS)
        def _():
            base = c * _N_ROWS
            pltpu.sync_copy(nb_hbm.at[pl.ds(c * _N_K, _N_K)], ni)
            for j in range(_N_K):
                pltpu.sync_copy(v_hbm.at[pl.ds(base + j * _N_W, _N_W)], gv)
                pltpu.sync_copy(gv, accu.at[ni.at[j]], add=True)

    plsc.subcore_barrier()
    pltpu.sync_copy(accu.at[pl.ds(sid * _GROWS, _GROWS)],
                    ou_hbm.at[cid, pl.ds(sid * _GROWS, _GROWS)])


@functools.partial(
    pl.kernel,
    out_type=(
        jax.ShapeDtypeStruct((2, N_V), jnp.float32),
        jax.ShapeDtypeStruct((2, N_G), jnp.float32),
        jax.ShapeDtypeStruct((2, N_G), jnp.float32),
    ),
    mesh=_MESH,
    compiler_params=pltpu.CompilerParams(use_tc_tiling_on_sc=False),
    scratch_types=[
        pltpu.VMEM_SHARED((N_V,), jnp.float32),
        pltpu.VMEM_SHARED((N_G,), jnp.float32),
        pltpu.VMEM_SHARED((N_G,), jnp.float32),
        pltpu.VMEM((_E_K, _E_W), jnp.int32),
        pltpu.VMEM((_E_K, _E_W), jnp.int32),
        pltpu.VMEM((_N_K, _N_W), jnp.int32),
        pltpu.VMEM((128,), jnp.float32),
    ],
)
def _sc_counts(dst_hbm, bb_hbm, nb_hbm, z_hbm, od_hbm, oe_hbm, ov_hbm,
               accd, acce, accv, di, bi, ni, ones):
    cid = lax.axis_index("c")
    sid = lax.axis_index("s")
    wid = _worker_id()
    for j in range(8):
        ones[pl.ds(j * 16, 16)] = jnp.ones((16,), jnp.float32)

    @pl.when(sid < _VSUB)
    def _():
        pltpu.sync_copy(z_hbm.at[pl.ds(sid * _VROWS, _VROWS)],
                        accd.at[pl.ds(sid * _VROWS, _VROWS)])

    pltpu.sync_copy(z_hbm.at[pl.ds(sid * _GROWS, _GROWS)],
                    acce.at[pl.ds(sid * _GROWS, _GROWS)])
    pltpu.sync_copy(z_hbm.at[pl.ds(sid * _GROWS, _GROWS)],
                    accv.at[pl.ds(sid * _GROWS, _GROWS)])
    plsc.subcore_barrier()

    @pl.loop(0, _E_TRIPS)
    def _(t):
        c = wid + t * _NW

        @pl.when(c < _E_CHUNKS)
        def _():
            pltpu.sync_copy(dst_hbm.at[pl.ds(c * _E_K, _E_K)], di)
            pltpu.sync_copy(bb_hbm.at[pl.ds(c * _E_K, _E_K)], bi)
            for j in range(_E_K):
                pltpu.sync_copy(ones.at[pl.ds(0, _E_W)], accd.at[di.at[j]],
                                add=True)
                pltpu.sync_copy(ones.at[pl.ds(0, _E_W)], acce.at[bi.at[j]],
                                add=True)

    @pl.loop(0, _N_TRIPS)
    def _(t):
        c = wid + t * _NW

        @pl.when(c < _N_CHUNKS)
        def _():
            pltpu.sync_copy(nb_hbm.at[pl.ds(c * _N_K, _N_K)], ni)
            for j in range(_N_K):
                pltpu.sync_copy(ones.at[pl.ds(0, _N_W)], accv.at[ni.at[j]],
                                add=True)

    plsc.subcore_barrier()

    @pl.when(sid < _VSUB)
    def _():
        pltpu.sync_copy(accd.at[pl.ds(sid * _VROWS, _VROWS)],
                        od_hbm.at[cid, pl.ds(sid * _VROWS, _VROWS)])

    pltpu.sync_copy(acce.at[pl.ds(sid * _GROWS, _GROWS)],
                    oe_hbm.at[cid, pl.ds(sid * _GROWS, _GROWS)])
    pltpu.sync_copy(accv.at[pl.ds(sid * _GROWS, _GROWS)],
                    ov_hbm.at[cid, pl.ds(sid * _GROWS, _GROWS)])


def _apply_mlp(h, layers):
    for p in layers:
        h = jax.nn.softplus(h @ p["W"] + p["b"])
    return h


def _head_kernel(tmp_ref, w0_ref, b0_ref, w1_ref, b1_ref, w2_ref, b2_ref, o_ref):
    h = jax.nn.softplus(tmp_ref[...] @ w0_ref[...] + b0_ref[...])
    h = jax.nn.softplus(h @ w1_ref[...] + b1_ref[...])
    o_ref[...] = h @ w2_ref[...] + b2_ref[...]


def _module(x, edge_attr, state, idx2d, p, recips):
    src2d, dst2d, bb2d, nb2d, zeros = idx2d
    rdeg, rce, rcv = recips
    xe = _apply_mlp(edge_attr, p["pre_e"])
    xv = _apply_mlp(x, p["pre_v"])
    xu = _apply_mlp(state, p["pre_u"])
    xvs, xvd, xub, xun = _sc_gather(xv, xu, src2d, dst2d, bb2d, nb2d)
    e_new = _apply_mlp(jnp.concatenate([xvs, xvd, xe, xub], axis=1), p["phi_e"])
    evp, uep = _sc_scatter_edges(e_new, dst2d, bb2d, zeros)
    e_to_v = (evp[0] + evp[1]) * rdeg
    v_new = _apply_mlp(jnp.concatenate([xv, e_to_v, xun], axis=1), p["phi_v"])
    uvp = _sc_scatter_nodes(v_new, nb2d, zeros)
    ue = (uep[0] + uep[1]) * rce
    uv = (uvp[0] + uvp[1]) * rcv
    u_new = _apply_mlp(jnp.concatenate([ue, uv, xu], axis=1), p["phi_u"])
    return v_new, e_new, u_new, uvp, uep


def kernel(x, edge_index, edge_attr, state, batch, bond_batch, params):
    src2d = edge_index[0].reshape(_E_CHUNKS * _E_K, _E_W)
    dst2d = edge_index[1].reshape(_E_CHUNKS * _E_K, _E_W)
    bb2d = bond_batch.reshape(_E_CHUNKS * _E_K, _E_W)
    nb2d = batch.reshape(_N_CHUNKS * _N_K, _N_W)
    zeros = jnp.zeros((N_V, D), jnp.float32)
    zeros1 = jnp.zeros((N_V,), jnp.float32)

    degp, cep, cvp = _sc_counts(dst2d, bb2d, nb2d, zeros1)
    rdeg = (1.0 / jnp.maximum(degp[0] + degp[1], 1.0))[:, None]
    rce = (1.0 / jnp.maximum(cep[0] + cep[1], 1.0))[:, None]
    rcv = (1.0 / jnp.maximum(cvp[0] + cvp[1], 1.0))[:, None]
    idx2d = (src2d, dst2d, bb2d, nb2d, zeros)
    recips = (rdeg, rce, rcv)

    v1, e1, u1, uvp1, uep1 = _module(x, edge_attr, state, idx2d, params["m1"], recips)
    x1, ee1, uu1 = v1, e1, u1
    v2, e2, u2, uvp2, uep2 = _module(x1, ee1, uu1, idx2d, params["m2"], recips)
    x2, ee2, uu2 = v2 + x1, e2 + ee1, u2 + uu1
    v3, e3, u3, uvp3, uep3 = _module(x2, ee2, uu2, idx2d, params["m3"], recips)
    uu3 = u3 + uu2

    mv_sum = (uvp1[0] + uvp1[1] + uvp2[0] + uvp2[1] + uvp3[0] + uvp3[1])
    me_sum = (uep1[0] + uep1[1] + uep2[0] + uep2[1] + uep3[0] + uep3[1])
    mv = mv_sum * rcv
    me = me_sum * rce
    z = jnp.zeros_like(mv)
    tmp = jnp.concatenate([z, mv, z, me, uu3], axis=1)
    hl = params["hiddens"]
    out = pl.pallas_call(
        _head_kernel,
        out_shape=jax.ShapeDtypeStruct((N_G, 1), jnp.float32),
    )(tmp, hl[0]["W"], hl[0]["b"], hl[1]["W"], hl[1]["b"], hl[2]["W"], hl[2]["b"])
    return out
